# Initial kernel scaffold; baseline (speedup 1.0000x reference)
import jax
import jax.numpy as jnp
from jax.experimental import pallas as pl


def kernel(circRNA, disease, cc_edge_index, dd_edge_index, cd_edge_index, tran_sample, lin_c_W, lin_d_W, cc_W0, cc_b0, cc_W1, cc_b1, dd_W0, dd_b0, dd_W1, dd_b1, cd_W0, cd_b0, cd_W1, cd_b1, mlp_W, mlp_b):
    raise NotImplementedError("write your pallas kernel here")


# speedup vs baseline: 16.2157x; 16.2157x over previous
import functools

import jax
import jax.numpy as jnp
from jax import lax
from jax.experimental import pallas as pl
from jax.experimental.pallas import tpu as pltpu
from jax.experimental.pallas import tpu_sc as plsc

CIRC = 1024
DIS = 512
CD = CIRC + DIS
NW = 32
ECC = 65536
EDD = 32768
ECD = 98304
NSAMP = 16384

_MESH = dict(core_axis_name="c", subcore_axis_name="s", num_cores=2,
             num_subcores=16)


def _adj_body(cc_src, cc_dst, dd_src, dd_dst, cd_src, cd_dst,
              acc_out, add_out, acd_out,
              buf_cc, buf_dd, buf_cd, se_v, de_v):
    wid = lax.axis_index("s") * 2 + lax.axis_index("c")
    ones = jnp.full((16,), 1.0, jnp.float32)
    zz = jnp.zeros((16,), jnp.float32)

    def build(src_hbm, dst_hbm, e_total, n, buf, out_hbm):
        rows = n // NW
        lo = wid * rows
        hi = lo + rows
        nwords = rows * n

        def zero_body(i, _):
            buf[pl.ds(i * 16, 16)] = zz
            return 0
        lax.fori_loop(0, nwords // 16, zero_body, 0)

        chunk = 2048

        def chunk_body(ci, _):
            base = ci * chunk
            pltpu.sync_copy(src_hbm.at[pl.ds(base, chunk)], se_v)
            pltpu.sync_copy(dst_hbm.at[pl.ds(base, chunk)], de_v)

            def vec_body(j, _):
                s = se_v[pl.ds(j * 16, 16)]
                dd = de_v[pl.ds(j * 16, 16)]
                m = (dd >= lo) & (dd < hi)
                idx = jnp.where(m, (dd - lo) * n + s, 0)
                plsc.addupdate_scatter(buf, [idx], ones, mask=m)
                return 0
            lax.fori_loop(0, chunk // 16, vec_body, 0)
            return 0
        lax.fori_loop(0, e_total // chunk, chunk_body, 0)
        pltpu.sync_copy(buf, out_hbm.at[pl.ds(lo * n, nwords)])

    build(cc_src, cc_dst, ECC, CIRC, buf_cc, acc_out)
    build(dd_src, dd_dst, EDD, DIS, buf_dd, add_out)
    build(cd_src, cd_dst, ECD, CD, buf_cd, acd_out)


_adj_kernel = functools.partial(
    pl.kernel,
    out_type=(
        jax.ShapeDtypeStruct((CIRC * CIRC,), jnp.float32),
        jax.ShapeDtypeStruct((DIS * DIS,), jnp.float32),
        jax.ShapeDtypeStruct((CD * CD,), jnp.float32),
    ),
    mesh=plsc.VectorSubcoreMesh(**_MESH),
    compiler_params=pltpu.CompilerParams(needs_layout_passes=False),
    scratch_types=(
        pltpu.VMEM(((CIRC // NW) * CIRC,), jnp.float32),
        pltpu.VMEM(((DIS // NW) * DIS,), jnp.float32),
        pltpu.VMEM(((CD // NW) * CD,), jnp.float32),
        pltpu.VMEM((2048,), jnp.int32),
        pltpu.VMEM((2048,), jnp.int32),
    ),
)(_adj_body)


def _mm(a, b):
    return lax.dot_general(a, b, (((1,), (0,)), ((), ())),
                           preferred_element_type=jnp.float32)


def _gcn(A, feat, W0, b0, W1, b1):
    deg = jnp.maximum(jnp.sum(A, axis=0), 1.0)
    norm = lax.rsqrt(deg)
    Ahat = A * norm[:, None] * norm[None, :]
    g0 = _mm(feat, W0)
    g1 = _mm(Ahat, g0)
    g2 = _mm(Ahat, g1)
    h0 = jnp.maximum(g0 + b0, 0.0)
    h1 = jnp.maximum(g1 + b0, 0.0)
    h2 = jnp.maximum(g2 + b0, 0.0)
    u0 = _mm(h0, W1[0:128]) + _mm(h1, W1[128:256]) + _mm(h2, W1[256:384])
    u1 = _mm(Ahat, u0)
    u2 = _mm(Ahat, u1)
    return (jnp.maximum(u0 + b1, 0.0), jnp.maximum(u1 + b1, 0.0),
            jnp.maximum(u2 + b1, 0.0))


def _dense_body(acc, add_, acd, circ, dis, lin_c, lin_d,
                cc_W0, cc_b0, cc_W1, cc_b1, dd_W0, dd_b0, dd_W1, dd_b1,
                cd_W0, cd_b0, cd_W1, cd_b1, mlp_W, mlp_b,
                sc_out, sd_out):
    cc = _gcn(acc[...], circ[...], cc_W0[...], cc_b0[...], cc_W1[...],
              cc_b1[...])
    dd = _gcn(add_[...], dis[...], dd_W0[...], dd_b0[...], dd_W1[...],
              dd_b1[...])
    cd_feat = jnp.concatenate(
        [_mm(circ[...], lin_c[...]), _mm(dis[...], lin_d[...])], axis=0)
    ass = _gcn(acd[...], cd_feat, cd_W0[...], cd_b0[...], cd_W1[...],
               cd_b1[...])
    W = mlp_W[...]
    score_c = _mm(cc[0], W[0:64]) + _mm(cc[1], W[64:128]) + _mm(cc[2], W[128:192])
    score_c = score_c + (_mm(ass[0][:CIRC], W[192:256]) +
                         _mm(ass[1][:CIRC], W[256:320]) +
                         _mm(ass[2][:CIRC], W[320:384]))
    score_d = _mm(dd[0], W[384:448]) + _mm(dd[1], W[448:512]) + _mm(dd[2], W[512:576])
    score_d = score_d + (_mm(ass[0][CIRC:], W[576:640]) +
                         _mm(ass[1][CIRC:], W[640:704]) +
                         _mm(ass[2][CIRC:], W[704:768]))
    sc_out[...] = score_c + mlp_b[...]
    sd_out[...] = score_d


def _pair_body(sc_hbm, sd_hbm, ts0_hbm, ts1_hbm, out_hbm,
               sc_v, sd_v, i0_v, i1_v, out_v):
    wid = lax.axis_index("s") * 2 + lax.axis_index("c")
    per = NSAMP // NW
    base = wid * per
    pltpu.sync_copy(sc_hbm, sc_v)
    pltpu.sync_copy(sd_hbm, sd_v)
    pltpu.sync_copy(ts0_hbm.at[pl.ds(base, per)], i0_v)
    pltpu.sync_copy(ts1_hbm.at[pl.ds(base, per)], i1_v)

    def body(j, _):
        i0 = i0_v[pl.ds(j * 16, 16)]
        i1 = i1_v[pl.ds(j * 16, 16)]
        v = plsc.load_gather(sc_v, [i0]) + plsc.load_gather(sd_v, [i1])
        out_v[pl.ds(j * 16, 16)] = 1.0 / (1.0 + jnp.exp(-v))
        return 0
    lax.fori_loop(0, per // 16, body, 0)
    pltpu.sync_copy(out_v, out_hbm.at[pl.ds(base, per)])


_pair_kernel = functools.partial(
    pl.kernel,
    out_type=jax.ShapeDtypeStruct((NSAMP,), jnp.float32),
    mesh=plsc.VectorSubcoreMesh(**_MESH),
    compiler_params=pltpu.CompilerParams(needs_layout_passes=False),
    scratch_types=(
        pltpu.VMEM((CIRC,), jnp.float32),
        pltpu.VMEM((DIS,), jnp.float32),
        pltpu.VMEM((NSAMP // NW,), jnp.int32),
        pltpu.VMEM((NSAMP // NW,), jnp.int32),
        pltpu.VMEM((NSAMP // NW,), jnp.float32),
    ),
)(_pair_body)


def kernel(circRNA, disease, cc_edge_index, dd_edge_index, cd_edge_index,
           tran_sample, lin_c_W, lin_d_W, cc_W0, cc_b0, cc_W1, cc_b1,
           dd_W0, dd_b0, dd_W1, dd_b1, cd_W0, cd_b0, cd_W1, cd_b1,
           mlp_W, mlp_b):
    acc, add_, acd = _adj_kernel(
        cc_edge_index[0], cc_edge_index[1],
        dd_edge_index[0], dd_edge_index[1],
        cd_edge_index[0], cd_edge_index[1])
    acc = acc.reshape(CIRC, CIRC)
    add_ = add_.reshape(DIS, DIS)
    acd = acd.reshape(CD, CD)

    score_c, score_d = pl.pallas_call(
        _dense_body,
        out_shape=(jax.ShapeDtypeStruct((CIRC, 1), jnp.float32),
                   jax.ShapeDtypeStruct((DIS, 1), jnp.float32)),
    )(acc, add_, acd, circRNA, disease, lin_c_W, lin_d_W,
      cc_W0, cc_b0.reshape(1, -1), cc_W1, cc_b1.reshape(1, -1),
      dd_W0, dd_b0.reshape(1, -1), dd_W1, dd_b1.reshape(1, -1),
      cd_W0, cd_b0.reshape(1, -1), cd_W1, cd_b1.reshape(1, -1),
      mlp_W, mlp_b.reshape(1, 1))

    out = _pair_kernel(score_c.reshape(CIRC), score_d.reshape(DIS),
                       tran_sample[:, 0], tran_sample[:, 1])
    return out.reshape(NSAMP, 1)


# speedup vs baseline: 29.0583x; 1.7920x over previous
import functools

import jax
import jax.numpy as jnp
from jax import lax
from jax.experimental import pallas as pl
from jax.experimental.pallas import tpu as pltpu
from jax.experimental.pallas import tpu_sc as plsc

CIRC = 1024
DIS = 512
CD = CIRC + DIS
NW = 32
ECC = 65536
EDD = 32768
ECD = 98304
NSAMP = 16384

_MESH = dict(core_axis_name="c", subcore_axis_name="s", num_cores=2,
             num_subcores=16)


_CH = 2048


def _adj_body(cc_src, cc_dst, dd_src, dd_dst, cd_src, cd_dst,
              acc_out, add_out, acd_out,
              buf_cc, buf_dd, buf_cd, se0, de0, se1, de1,
              sem0, sem1, osem):
    wid = lax.axis_index("s") * 2 + lax.axis_index("c")
    ones = jnp.full((16,), 1.0, jnp.float32)
    zz = jnp.zeros((16,), jnp.float32)

    def build(src_hbm, dst_hbm, e_total, n, buf, out_hbm):
        rows = n // NW
        lo = wid * rows
        hi = lo + rows
        nwords = rows * n

        def zero_body(i, _):
            for u in range(8):
                buf[pl.ds((i * 8 + u) * 16, 16)] = zz
            return 0
        lax.fori_loop(0, nwords // 128, zero_body, 0)

        def start_load(ci, sbuf, dbuf, sem):
            base = ci * _CH
            pltpu.async_copy(src_hbm.at[pl.ds(base, _CH)], sbuf, sem)
            pltpu.async_copy(dst_hbm.at[pl.ds(base, _CH)], dbuf, sem)

        def wait_load(sbuf, dbuf, sem):
            pltpu.make_async_copy(src_hbm.at[pl.ds(0, _CH)], sbuf, sem).wait()
            pltpu.make_async_copy(dst_hbm.at[pl.ds(0, _CH)], dbuf, sem).wait()

        def scan(sbuf, dbuf):
            def grp_body(j, _):
                for u in range(4):
                    off = j * 64 + u * 16
                    s = sbuf[pl.ds(off, 16)]
                    dd = dbuf[pl.ds(off, 16)]
                    m = (dd >= lo) & (dd < hi)
                    idx = jnp.where(m, (dd - lo) * n + s, 0)
                    plsc.addupdate_scatter(buf, [idx], ones, mask=m)
                return 0
            lax.fori_loop(0, _CH // 64, grp_body, 0)

        nch = e_total // _CH
        start_load(0, se0, de0, sem0)

        def pair_body(pi, _):
            c0 = pi * 2
            start_load(c0 + 1, se1, de1, sem1)
            wait_load(se0, de0, sem0)
            scan(se0, de0)

            @pl.when(c0 + 2 < nch)
            def _():
                start_load(c0 + 2, se0, de0, sem0)
            wait_load(se1, de1, sem1)
            scan(se1, de1)
            return 0
        lax.fori_loop(0, nch // 2, pair_body, 0)
        pltpu.async_copy(buf, out_hbm.at[pl.ds(lo * n, nwords)], osem)

    def drain(n, buf, out_hbm):
        rows = n // NW
        lo = wid * rows
        pltpu.make_async_copy(
            buf, out_hbm.at[pl.ds(lo * n, rows * n)], osem).wait()

    build(cc_src, cc_dst, ECC, CIRC, buf_cc, acc_out)
    build(dd_src, dd_dst, EDD, DIS, buf_dd, add_out)
    build(cd_src, cd_dst, ECD, CD, buf_cd, acd_out)
    drain(CIRC, buf_cc, acc_out)
    drain(DIS, buf_dd, add_out)
    drain(CD, buf_cd, acd_out)


_adj_kernel = functools.partial(
    pl.kernel,
    out_type=(
        jax.ShapeDtypeStruct((CIRC * CIRC,), jnp.float32),
        jax.ShapeDtypeStruct((DIS * DIS,), jnp.float32),
        jax.ShapeDtypeStruct((CD * CD,), jnp.float32),
    ),
    mesh=plsc.VectorSubcoreMesh(**_MESH),
    compiler_params=pltpu.CompilerParams(needs_layout_passes=False),
    scratch_types=(
        pltpu.VMEM(((CIRC // NW) * CIRC,), jnp.float32),
        pltpu.VMEM(((DIS // NW) * DIS,), jnp.float32),
        pltpu.VMEM(((CD // NW) * CD,), jnp.float32),
        pltpu.VMEM((_CH,), jnp.int32),
        pltpu.VMEM((_CH,), jnp.int32),
        pltpu.VMEM((_CH,), jnp.int32),
        pltpu.VMEM((_CH,), jnp.int32),
        pltpu.SemaphoreType.DMA,
        pltpu.SemaphoreType.DMA,
        pltpu.SemaphoreType.DMA,
    ),
)(_adj_body)


def _mm(a, b):
    return lax.dot_general(a, b, (((1,), (0,)), ((), ())),
                           preferred_element_type=jnp.float32)


def _gcn(A, feat, W0, b0, W1, b1):
    deg = jnp.maximum(jnp.sum(A, axis=0), 1.0)
    norm = lax.rsqrt(deg)
    Ahat = A * norm[:, None] * norm[None, :]
    g0 = _mm(feat, W0)
    g1 = _mm(Ahat, g0)
    g2 = _mm(Ahat, g1)
    h0 = jnp.maximum(g0 + b0, 0.0)
    h1 = jnp.maximum(g1 + b0, 0.0)
    h2 = jnp.maximum(g2 + b0, 0.0)
    u0 = _mm(h0, W1[0:128]) + _mm(h1, W1[128:256]) + _mm(h2, W1[256:384])
    u1 = _mm(Ahat, u0)
    u2 = _mm(Ahat, u1)
    return (jnp.maximum(u0 + b1, 0.0), jnp.maximum(u1 + b1, 0.0),
            jnp.maximum(u2 + b1, 0.0))


def _dense_body(acc, add_, acd, circ, dis, lin_c, lin_d,
                cc_W0, cc_b0, cc_W1, cc_b1, dd_W0, dd_b0, dd_W1, dd_b1,
                cd_W0, cd_b0, cd_W1, cd_b1, mlp_W, mlp_b,
                sc_out, sd_out):
    cc = _gcn(acc[...], circ[...], cc_W0[...], cc_b0[...], cc_W1[...],
              cc_b1[...])
    dd = _gcn(add_[...], dis[...], dd_W0[...], dd_b0[...], dd_W1[...],
              dd_b1[...])
    cd_feat = jnp.concatenate(
        [_mm(circ[...], lin_c[...]), _mm(dis[...], lin_d[...])], axis=0)
    ass = _gcn(acd[...], cd_feat, cd_W0[...], cd_b0[...], cd_W1[...],
               cd_b1[...])
    W = mlp_W[...]
    score_c = _mm(cc[0], W[0:64]) + _mm(cc[1], W[64:128]) + _mm(cc[2], W[128:192])
    score_c = score_c + (_mm(ass[0][:CIRC], W[192:256]) +
                         _mm(ass[1][:CIRC], W[256:320]) +
                         _mm(ass[2][:CIRC], W[320:384]))
    score_d = _mm(dd[0], W[384:448]) + _mm(dd[1], W[448:512]) + _mm(dd[2], W[512:576])
    score_d = score_d + (_mm(ass[0][CIRC:], W[576:640]) +
                         _mm(ass[1][CIRC:], W[640:704]) +
                         _mm(ass[2][CIRC:], W[704:768]))
    sc_out[...] = score_c + mlp_b[...]
    sd_out[...] = score_d


def _pair_body(sc_hbm, sd_hbm, ts0_hbm, ts1_hbm, out_hbm,
               sc_v, sd_v, i0_v, i1_v, out_v):
    wid = lax.axis_index("s") * 2 + lax.axis_index("c")
    per = NSAMP // NW
    base = wid * per
    pltpu.sync_copy(sc_hbm, sc_v)
    pltpu.sync_copy(sd_hbm, sd_v)
    pltpu.sync_copy(ts0_hbm.at[pl.ds(base, per)], i0_v)
    pltpu.sync_copy(ts1_hbm.at[pl.ds(base, per)], i1_v)

    def body(j, _):
        i0 = i0_v[pl.ds(j * 16, 16)]
        i1 = i1_v[pl.ds(j * 16, 16)]
        v = plsc.load_gather(sc_v, [i0]) + plsc.load_gather(sd_v, [i1])
        out_v[pl.ds(j * 16, 16)] = 1.0 / (1.0 + jnp.exp(-v))
        return 0
    lax.fori_loop(0, per // 16, body, 0)
    pltpu.sync_copy(out_v, out_hbm.at[pl.ds(base, per)])


_pair_kernel = functools.partial(
    pl.kernel,
    out_type=jax.ShapeDtypeStruct((NSAMP,), jnp.float32),
    mesh=plsc.VectorSubcoreMesh(**_MESH),
    compiler_params=pltpu.CompilerParams(needs_layout_passes=False),
    scratch_types=(
        pltpu.VMEM((CIRC,), jnp.float32),
        pltpu.VMEM((DIS,), jnp.float32),
        pltpu.VMEM((NSAMP // NW,), jnp.int32),
        pltpu.VMEM((NSAMP // NW,), jnp.int32),
        pltpu.VMEM((NSAMP // NW,), jnp.float32),
    ),
)(_pair_body)


def kernel(circRNA, disease, cc_edge_index, dd_edge_index, cd_edge_index,
           tran_sample, lin_c_W, lin_d_W, cc_W0, cc_b0, cc_W1, cc_b1,
           dd_W0, dd_b0, dd_W1, dd_b1, cd_W0, cd_b0, cd_W1, cd_b1,
           mlp_W, mlp_b):
    acc, add_, acd = _adj_kernel(
        cc_edge_index[0], cc_edge_index[1],
        dd_edge_index[0], dd_edge_index[1],
        cd_edge_index[0], cd_edge_index[1])
    acc = acc.reshape(CIRC, CIRC)
    add_ = add_.reshape(DIS, DIS)
    acd = acd.reshape(CD, CD)

    score_c, score_d = pl.pallas_call(
        _dense_body,
        out_shape=(jax.ShapeDtypeStruct((CIRC, 1), jnp.float32),
                   jax.ShapeDtypeStruct((DIS, 1), jnp.float32)),
    )(acc, add_, acd, circRNA, disease, lin_c_W, lin_d_W,
      cc_W0, cc_b0.reshape(1, -1), cc_W1, cc_b1.reshape(1, -1),
      dd_W0, dd_b0.reshape(1, -1), dd_W1, dd_b1.reshape(1, -1),
      cd_W0, cd_b0.reshape(1, -1), cd_W1, cd_b1.reshape(1, -1),
      mlp_W, mlp_b.reshape(1, 1))

    out = _pair_kernel(score_c.reshape(CIRC), score_d.reshape(DIS),
                       tran_sample[:, 0], tran_sample[:, 1])
    return out.reshape(NSAMP, 1)


# speedup vs baseline: 43.9260x; 1.5117x over previous
import functools

import jax
import jax.numpy as jnp
from jax import lax
from jax.experimental import pallas as pl
from jax.experimental.pallas import tpu as pltpu
from jax.experimental.pallas import tpu_sc as plsc

CIRC = 1024
DIS = 512
CD = CIRC + DIS
NW = 32
ECC = 65536
EDD = 32768
ECD = 98304
NSAMP = 16384

_MESH = dict(core_axis_name="c", subcore_axis_name="s", num_cores=2,
             num_subcores=16)


_TCC = ECC // 16
_TDD = EDD // 16
_TCD = ECD // 16
_NGRP = (_TCC + _TDD + _TCD) // 16
_NROW = _NGRP // 8
_RCC = _TCC // 128
_RDD = _TDD // 128
_RCD = _TCD // 128
_TAIL = 2048
_ZCH = 8192


def _adj_body(cc_src, cc_dst, dd_src, dd_dst, cd_src, cd_dst,
              acc_out, add_out, acd_out,
              es_cc, ed_cc, es_dd, ed_dd, es_cd, ed_cd,
              idxb, valb, zbuf,
              scc, sdd, scd,
              lsem, zsem, ssem, osem):
    c = lax.axis_index("c")
    sid = lax.axis_index("s")
    zz = jnp.zeros((16,), jnp.int32)

    for hbm, vmem, cnt in ((cc_src, es_cc, _TCC), (cc_dst, ed_cc, _TCC),
                           (dd_src, es_dd, _TDD), (dd_dst, ed_dd, _TDD),
                           (cd_src, es_cd, _TCD), (cd_dst, ed_cd, _TCD)):
        pltpu.async_copy(hbm.at[pl.ds(sid * cnt, cnt)], vmem, lsem)

    def zb(i, _):
        for u in range(8):
            zbuf[pl.ds((i * 8 + u) * 16, 16)] = zz
        return 0
    lax.fori_loop(0, _ZCH // 128, zb, 0)

    def zero_plan(half, n):
        share = (half * (n // 2) + _TAIL) // 16
        chunks = []
        done = 0
        while done < share:
            sz = min(_ZCH, share - done)
            chunks.append((done, sz))
            done += sz
        return share, chunks

    for shared, half, n in ((scc, CIRC // 2, CIRC), (sdd, DIS // 2, DIS),
                            (scd, CD // 2, CD)):
        share, chunks = zero_plan(half, n)
        for off, sz in chunks:
            pltpu.async_copy(zbuf.at[pl.ds(0, sz)],
                             shared.at[pl.ds(sid * share + off, sz)], zsem)

    for hbm, vmem, cnt in ((cc_src, es_cc, _TCC), (cc_dst, ed_cc, _TCC),
                           (dd_src, es_dd, _TDD), (dd_dst, ed_dd, _TDD),
                           (cd_src, es_cd, _TCD), (cd_dst, ed_cd, _TCD)):
        pltpu.make_async_copy(hbm.at[pl.ds(0, cnt)], vmem, lsem).wait()

    def scan(es, ed, n, half, row0, nrows):
        base = c * half
        nw = n // 2
        dump = half * nw

        def body(r, _):
            for u in range(8):
                off = r * 128 + u * 16
                s16 = es[pl.ds(off, 16)]
                d16 = ed[pl.ds(off, 16)]
                m = (d16 >= base) & (d16 < base + half)
                hi_half = s16 >= nw
                sp = jnp.where(hi_half, s16 - nw, s16)
                val = jnp.where(hi_half, 65536, 1)
                idx = jnp.where(m, (d16 - base) * nw + sp, dump)
                idxb[row0 + r, pl.ds(u * 16, 16)] = idx
                valb[row0 + r, pl.ds(u * 16, 16)] = val
            return 0
        lax.fori_loop(0, nrows, body, 0)

    scan(es_cc, ed_cc, CIRC, CIRC // 2, 0, _RCC)
    scan(es_dd, ed_dd, DIS, DIS // 2, _RCC, _RDD)
    scan(es_cd, ed_cd, CD, CD // 2, _RCC + _RDD, _RCD)

    for shared, half, n in ((scc, CIRC // 2, CIRC), (sdd, DIS // 2, DIS),
                            (scd, CD // 2, CD)):
        share, chunks = zero_plan(half, n)
        for off, sz in chunks:
            pltpu.make_async_copy(zbuf.at[pl.ds(0, sz)],
                                  shared.at[pl.ds(0, sz)], zsem).wait()
    plsc.subcore_barrier()

    def fire_scatter(shared, row0, nrows):
        def body(j, _):
            pltpu.async_copy(valb.at[row0 + j],
                             shared.at[idxb.at[row0 + j]], ssem, add=True)
            return 0
        lax.fori_loop(0, nrows, body, 0)

    fire_scatter(scc, 0, _RCC)
    fire_scatter(sdd, _RCC, _RDD)
    fire_scatter(scd, _RCC + _RDD, _RCD)

    def swait(shared, row0, nrows):
        def body(j, _):
            pltpu.make_async_copy(valb.at[row0 + j],
                                  shared.at[idxb.at[row0 + j]], ssem).wait()
            return 0
        lax.fori_loop(0, nrows, body, 0)

    swait(scc, 0, _RCC)
    swait(sdd, _RCC, _RDD)
    swait(scd, _RCC + _RDD, _RCD)
    plsc.subcore_barrier()

    for shared, half, n, out in ((scc, CIRC // 2, CIRC, acc_out),
                                 (sdd, DIS // 2, DIS, add_out),
                                 (scd, CD // 2, CD, acd_out)):
        share = half * (n // 2) // 16
        pltpu.async_copy(shared.at[pl.ds(sid * share, share)],
                         out.at[pl.ds(c * half * (n // 2) + sid * share,
                                      share)], osem)
    for shared, half, n, out in ((scc, CIRC // 2, CIRC, acc_out),
                                 (sdd, DIS // 2, DIS, add_out),
                                 (scd, CD // 2, CD, acd_out)):
        share = half * (n // 2) // 16
        pltpu.make_async_copy(shared.at[pl.ds(0, share)],
                              out.at[pl.ds(0, share)], osem).wait()


_adj_kernel = functools.partial(
    pl.kernel,
    out_type=(
        jax.ShapeDtypeStruct((CIRC * (CIRC // 2),), jnp.int32),
        jax.ShapeDtypeStruct((DIS * (DIS // 2),), jnp.int32),
        jax.ShapeDtypeStruct((CD * (CD // 2),), jnp.int32),
    ),
    mesh=plsc.VectorSubcoreMesh(**_MESH),
    compiler_params=pltpu.CompilerParams(needs_layout_passes=False),
    scratch_types=(
        pltpu.VMEM((_TCC,), jnp.int32),
        pltpu.VMEM((_TCC,), jnp.int32),
        pltpu.VMEM((_TDD,), jnp.int32),
        pltpu.VMEM((_TDD,), jnp.int32),
        pltpu.VMEM((_TCD,), jnp.int32),
        pltpu.VMEM((_TCD,), jnp.int32),
        pltpu.VMEM((_NROW, 128), jnp.int32),
        pltpu.VMEM((_NROW, 128), jnp.int32),
        pltpu.VMEM((_ZCH,), jnp.int32),
        pltpu.VMEM_SHARED(((CIRC // 2) * (CIRC // 2) + _TAIL,), jnp.int32),
        pltpu.VMEM_SHARED(((DIS // 2) * (DIS // 2) + _TAIL,), jnp.int32),
        pltpu.VMEM_SHARED(((CD // 2) * (CD // 2) + _TAIL,), jnp.int32),
        pltpu.SemaphoreType.DMA,
        pltpu.SemaphoreType.DMA,
        pltpu.SemaphoreType.DMA,
        pltpu.SemaphoreType.DMA,
    ),
)(_adj_body)


def _mm(a, b):
    return lax.dot_general(a, b, (((1,), (0,)), ((), ())),
                           preferred_element_type=jnp.float32)


def _gcn(P, feat, W0, b0, W1, b1):
    lo = (P & 0xFFFF).astype(jnp.float32)
    hi = lax.shift_right_logical(P, 16).astype(jnp.float32)
    A = jnp.concatenate([lo, hi], axis=1)
    deg = jnp.maximum(jnp.sum(A, axis=0), 1.0)
    norm = lax.rsqrt(deg)
    Ahat = A * norm[:, None] * norm[None, :]
    g0 = _mm(feat, W0)
    g1 = _mm(Ahat, g0)
    g2 = _mm(Ahat, g1)
    h0 = jnp.maximum(g0 + b0, 0.0)
    h1 = jnp.maximum(g1 + b0, 0.0)
    h2 = jnp.maximum(g2 + b0, 0.0)
    u0 = _mm(h0, W1[0:128]) + _mm(h1, W1[128:256]) + _mm(h2, W1[256:384])
    u1 = _mm(Ahat, u0)
    u2 = _mm(Ahat, u1)
    return (jnp.maximum(u0 + b1, 0.0), jnp.maximum(u1 + b1, 0.0),
            jnp.maximum(u2 + b1, 0.0))


def _dense_body(acc, add_, acd, circ, dis, lin_c, lin_d,
                cc_W0, cc_b0, cc_W1, cc_b1, dd_W0, dd_b0, dd_W1, dd_b1,
                cd_W0, cd_b0, cd_W1, cd_b1, mlp_W, mlp_b,
                sc_out, sd_out):
    cc = _gcn(acc[...], circ[...], cc_W0[...], cc_b0[...], cc_W1[...],
              cc_b1[...])
    dd = _gcn(add_[...], dis[...], dd_W0[...], dd_b0[...], dd_W1[...],
              dd_b1[...])
    cd_feat = jnp.concatenate(
        [_mm(circ[...], lin_c[...]), _mm(dis[...], lin_d[...])], axis=0)
    ass = _gcn(acd[...], cd_feat, cd_W0[...], cd_b0[...], cd_W1[...],
               cd_b1[...])
    W = mlp_W[...]
    score_c = _mm(cc[0], W[0:64]) + _mm(cc[1], W[64:128]) + _mm(cc[2], W[128:192])
    score_c = score_c + (_mm(ass[0][:CIRC], W[192:256]) +
                         _mm(ass[1][:CIRC], W[256:320]) +
                         _mm(ass[2][:CIRC], W[320:384]))
    score_d = _mm(dd[0], W[384:448]) + _mm(dd[1], W[448:512]) + _mm(dd[2], W[512:576])
    score_d = score_d + (_mm(ass[0][CIRC:], W[576:640]) +
                         _mm(ass[1][CIRC:], W[640:704]) +
                         _mm(ass[2][CIRC:], W[704:768]))
    sc_out[...] = score_c + mlp_b[...]
    sd_out[...] = score_d


def _pair_body(sc_hbm, sd_hbm, ts0_hbm, ts1_hbm, out_hbm,
               sc_v, sd_v, i0_v, i1_v, out_v):
    wid = lax.axis_index("s") * 2 + lax.axis_index("c")
    per = NSAMP // NW
    base = wid * per
    pltpu.sync_copy(sc_hbm, sc_v)
    pltpu.sync_copy(sd_hbm, sd_v)
    pltpu.sync_copy(ts0_hbm.at[pl.ds(base, per)], i0_v)
    pltpu.sync_copy(ts1_hbm.at[pl.ds(base, per)], i1_v)

    def body(j, _):
        i0 = i0_v[pl.ds(j * 16, 16)]
        i1 = i1_v[pl.ds(j * 16, 16)]
        v = plsc.load_gather(sc_v, [i0]) + plsc.load_gather(sd_v, [i1])
        out_v[pl.ds(j * 16, 16)] = 1.0 / (1.0 + jnp.exp(-v))
        return 0
    lax.fori_loop(0, per // 16, body, 0)
    pltpu.sync_copy(out_v, out_hbm.at[pl.ds(base, per)])


_pair_kernel = functools.partial(
    pl.kernel,
    out_type=jax.ShapeDtypeStruct((NSAMP,), jnp.float32),
    mesh=plsc.VectorSubcoreMesh(**_MESH),
    compiler_params=pltpu.CompilerParams(needs_layout_passes=False),
    scratch_types=(
        pltpu.VMEM((CIRC,), jnp.float32),
        pltpu.VMEM((DIS,), jnp.float32),
        pltpu.VMEM((NSAMP // NW,), jnp.int32),
        pltpu.VMEM((NSAMP // NW,), jnp.int32),
        pltpu.VMEM((NSAMP // NW,), jnp.float32),
    ),
)(_pair_body)


def kernel(circRNA, disease, cc_edge_index, dd_edge_index, cd_edge_index,
           tran_sample, lin_c_W, lin_d_W, cc_W0, cc_b0, cc_W1, cc_b1,
           dd_W0, dd_b0, dd_W1, dd_b1, cd_W0, cd_b0, cd_W1, cd_b1,
           mlp_W, mlp_b):
    acc, add_, acd = _adj_kernel(
        cc_edge_index[0], cc_edge_index[1],
        dd_edge_index[0], dd_edge_index[1],
        cd_edge_index[0], cd_edge_index[1])
    acc = acc.reshape(CIRC, CIRC // 2)
    add_ = add_.reshape(DIS, DIS // 2)
    acd = acd.reshape(CD, CD // 2)

    score_c, score_d = pl.pallas_call(
        _dense_body,
        out_shape=(jax.ShapeDtypeStruct((CIRC, 1), jnp.float32),
                   jax.ShapeDtypeStruct((DIS, 1), jnp.float32)),
    )(acc, add_, acd, circRNA, disease, lin_c_W, lin_d_W,
      cc_W0, cc_b0.reshape(1, -1), cc_W1, cc_b1.reshape(1, -1),
      dd_W0, dd_b0.reshape(1, -1), dd_W1, dd_b1.reshape(1, -1),
      cd_W0, cd_b0.reshape(1, -1), cd_W1, cd_b1.reshape(1, -1),
      mlp_W, mlp_b.reshape(1, 1))

    out = _pair_kernel(score_c.reshape(CIRC), score_d.reshape(DIS),
                       tran_sample[:, 0], tran_sample[:, 1])
    return out.reshape(NSAMP, 1)


# speedup vs baseline: 44.1055x; 1.0041x over previous
import functools

import jax
import jax.numpy as jnp
from jax import lax
from jax.experimental import pallas as pl
from jax.experimental.pallas import tpu as pltpu
from jax.experimental.pallas import tpu_sc as plsc

CIRC = 1024
DIS = 512
CD = CIRC + DIS
NW = 32
ECC = 65536
EDD = 32768
ECD = 98304
NSAMP = 16384

_MESH = dict(core_axis_name="c", subcore_axis_name="s", num_cores=2,
             num_subcores=16)


_TCC = ECC // 16
_TDD = EDD // 16
_TCD = ECD // 16
_NGRP = (_TCC + _TDD + _TCD) // 16
_NROW = _NGRP // 8
_RCC = _TCC // 128
_RDD = _TDD // 128
_RCD = _TCD // 128
_TAIL = 2048
_ZCH = 8192


def _adj_body(cc_ei, dd_ei, cd_ei,
              acc_out, add_out, acd_out,
              es_cc, ed_cc, es_dd, ed_dd, es_cd, ed_cd,
              idxb, valb, zbuf,
              scc, sdd, scd,
              lsem, zsem, ssem, osem):
    c = lax.axis_index("c")
    sid = lax.axis_index("s")
    zz = jnp.zeros((16,), jnp.int32)

    for hbm, row, vmem, cnt in ((cc_ei, 0, es_cc, _TCC), (cc_ei, 1, ed_cc, _TCC),
                                (dd_ei, 0, es_dd, _TDD), (dd_ei, 1, ed_dd, _TDD),
                                (cd_ei, 0, es_cd, _TCD), (cd_ei, 1, ed_cd, _TCD)):
        pltpu.async_copy(hbm.at[pl.ds(row, 1), pl.ds(sid * cnt, cnt)],
                         vmem, lsem)

    def zb(i, _):
        for u in range(8):
            zbuf[pl.ds((i * 8 + u) * 16, 16)] = zz
        return 0
    lax.fori_loop(0, _ZCH // 128, zb, 0)

    def zero_plan(half, n):
        share = (half * (n // 2) + _TAIL) // 16
        chunks = []
        done = 0
        while done < share:
            sz = min(_ZCH, share - done)
            chunks.append((done, sz))
            done += sz
        return share, chunks

    for shared, half, n in ((scc, CIRC // 2, CIRC), (sdd, DIS // 2, DIS),
                            (scd, CD // 2, CD)):
        share, chunks = zero_plan(half, n)
        for off, sz in chunks:
            pltpu.async_copy(zbuf.at[pl.ds(0, sz)],
                             shared.at[pl.ds(sid * share + off, sz)], zsem)

    for hbm, row, vmem, cnt in ((cc_ei, 0, es_cc, _TCC), (cc_ei, 1, ed_cc, _TCC),
                                (dd_ei, 0, es_dd, _TDD), (dd_ei, 1, ed_dd, _TDD),
                                (cd_ei, 0, es_cd, _TCD), (cd_ei, 1, ed_cd, _TCD)):
        pltpu.make_async_copy(hbm.at[pl.ds(row, 1), pl.ds(0, cnt)], vmem,
                              lsem).wait()

    def scan(es, ed, n, half, row0, nrows):
        base = c * half
        nw = n // 2
        dump = half * nw

        def body(r, _):
            for u in range(8):
                off = r * 128 + u * 16
                s16 = es[0, pl.ds(off, 16)]
                d16 = ed[0, pl.ds(off, 16)]
                m = (d16 >= base) & (d16 < base + half)
                hi_half = s16 >= nw
                sp = jnp.where(hi_half, s16 - nw, s16)
                val = jnp.where(hi_half, 65536, 1)
                idx = jnp.where(m, (d16 - base) * nw + sp, dump)
                idxb[row0 + r, pl.ds(u * 16, 16)] = idx
                valb[row0 + r, pl.ds(u * 16, 16)] = val
            return 0
        lax.fori_loop(0, nrows, body, 0)

    scan(es_cc, ed_cc, CIRC, CIRC // 2, 0, _RCC)
    scan(es_dd, ed_dd, DIS, DIS // 2, _RCC, _RDD)
    scan(es_cd, ed_cd, CD, CD // 2, _RCC + _RDD, _RCD)

    for shared, half, n in ((scc, CIRC // 2, CIRC), (sdd, DIS // 2, DIS),
                            (scd, CD // 2, CD)):
        share, chunks = zero_plan(half, n)
        for off, sz in chunks:
            pltpu.make_async_copy(zbuf.at[pl.ds(0, sz)],
                                  shared.at[pl.ds(0, sz)], zsem).wait()
    plsc.subcore_barrier()

    def fire_scatter(shared, row0, nrows):
        def body(j, _):
            pltpu.async_copy(valb.at[row0 + j],
                             shared.at[idxb.at[row0 + j]], ssem, add=True)
            return 0
        lax.fori_loop(0, nrows, body, 0)

    fire_scatter(scc, 0, _RCC)
    fire_scatter(sdd, _RCC, _RDD)
    fire_scatter(scd, _RCC + _RDD, _RCD)

    def swait(shared, row0, nrows):
        def body(j, _):
            pltpu.make_async_copy(valb.at[row0 + j],
                                  shared.at[idxb.at[row0 + j]], ssem).wait()
            return 0
        lax.fori_loop(0, nrows, body, 0)

    swait(scc, 0, _RCC)
    swait(sdd, _RCC, _RDD)
    swait(scd, _RCC + _RDD, _RCD)
    plsc.subcore_barrier()

    for shared, half, n, out in ((scc, CIRC // 2, CIRC, acc_out),
                                 (sdd, DIS // 2, DIS, add_out),
                                 (scd, CD // 2, CD, acd_out)):
        share = half * (n // 2) // 16
        pltpu.async_copy(shared.at[pl.ds(sid * share, share)],
                         out.at[pl.ds(c * half * (n // 2) + sid * share,
                                      share)], osem)
    for shared, half, n, out in ((scc, CIRC // 2, CIRC, acc_out),
                                 (sdd, DIS // 2, DIS, add_out),
                                 (scd, CD // 2, CD, acd_out)):
        share = half * (n // 2) // 16
        pltpu.make_async_copy(shared.at[pl.ds(0, share)],
                              out.at[pl.ds(0, share)], osem).wait()


_adj_kernel = functools.partial(
    pl.kernel,
    out_type=(
        jax.ShapeDtypeStruct((CIRC * (CIRC // 2),), jnp.int32),
        jax.ShapeDtypeStruct((DIS * (DIS // 2),), jnp.int32),
        jax.ShapeDtypeStruct((CD * (CD // 2),), jnp.int32),
    ),
    mesh=plsc.VectorSubcoreMesh(**_MESH),
    compiler_params=pltpu.CompilerParams(needs_layout_passes=False),
    scratch_types=(
        pltpu.VMEM((1, _TCC), jnp.int32),
        pltpu.VMEM((1, _TCC), jnp.int32),
        pltpu.VMEM((1, _TDD), jnp.int32),
        pltpu.VMEM((1, _TDD), jnp.int32),
        pltpu.VMEM((1, _TCD), jnp.int32),
        pltpu.VMEM((1, _TCD), jnp.int32),
        pltpu.VMEM((_NROW, 128), jnp.int32),
        pltpu.VMEM((_NROW, 128), jnp.int32),
        pltpu.VMEM((_ZCH,), jnp.int32),
        pltpu.VMEM_SHARED(((CIRC // 2) * (CIRC // 2) + _TAIL,), jnp.int32),
        pltpu.VMEM_SHARED(((DIS // 2) * (DIS // 2) + _TAIL,), jnp.int32),
        pltpu.VMEM_SHARED(((CD // 2) * (CD // 2) + _TAIL,), jnp.int32),
        pltpu.SemaphoreType.DMA,
        pltpu.SemaphoreType.DMA,
        pltpu.SemaphoreType.DMA,
        pltpu.SemaphoreType.DMA,
    ),
)(_adj_body)


def _mm(a, b):
    return lax.dot_general(a, b, (((1,), (0,)), ((), ())),
                           preferred_element_type=jnp.float32)


def _gcn(P, feat, W0, b0, W1, b1):
    lo = (P & 0xFFFF).astype(jnp.float32)
    hi = lax.shift_right_logical(P, 16).astype(jnp.float32)
    A = jnp.concatenate([lo, hi], axis=1)
    deg = jnp.maximum(jnp.sum(A, axis=0), 1.0)
    norm = lax.rsqrt(deg)
    Ahat = A * norm[:, None] * norm[None, :]
    g0 = _mm(feat, W0)
    g1 = _mm(Ahat, g0)
    g2 = _mm(Ahat, g1)
    h0 = jnp.maximum(g0 + b0, 0.0)
    h1 = jnp.maximum(g1 + b0, 0.0)
    h2 = jnp.maximum(g2 + b0, 0.0)
    u0 = _mm(h0, W1[0:128]) + _mm(h1, W1[128:256]) + _mm(h2, W1[256:384])
    u1 = _mm(Ahat, u0)
    u2 = _mm(Ahat, u1)
    return (jnp.maximum(u0 + b1, 0.0), jnp.maximum(u1 + b1, 0.0),
            jnp.maximum(u2 + b1, 0.0))


def _dense_body(acc, add_, acd, circ, dis, lin_c, lin_d,
                cc_W0, cc_b0, cc_W1, cc_b1, dd_W0, dd_b0, dd_W1, dd_b1,
                cd_W0, cd_b0, cd_W1, cd_b1, mlp_W, mlp_b,
                sc_out, sd_out):
    cc = _gcn(acc[...], circ[...], cc_W0[...], cc_b0[...], cc_W1[...],
              cc_b1[...])
    dd = _gcn(add_[...], dis[...], dd_W0[...], dd_b0[...], dd_W1[...],
              dd_b1[...])
    cd_feat = jnp.concatenate(
        [_mm(circ[...], lin_c[...]), _mm(dis[...], lin_d[...])], axis=0)
    ass = _gcn(acd[...], cd_feat, cd_W0[...], cd_b0[...], cd_W1[...],
               cd_b1[...])
    W = mlp_W[...]
    score_c = _mm(cc[0], W[0:64]) + _mm(cc[1], W[64:128]) + _mm(cc[2], W[128:192])
    score_c = score_c + (_mm(ass[0][:CIRC], W[192:256]) +
                         _mm(ass[1][:CIRC], W[256:320]) +
                         _mm(ass[2][:CIRC], W[320:384]))
    score_d = _mm(dd[0], W[384:448]) + _mm(dd[1], W[448:512]) + _mm(dd[2], W[512:576])
    score_d = score_d + (_mm(ass[0][CIRC:], W[576:640]) +
                         _mm(ass[1][CIRC:], W[640:704]) +
                         _mm(ass[2][CIRC:], W[704:768]))
    sc_out[...] = score_c + mlp_b[...]
    sd_out[...] = score_d


def _pair_body(sc_hbm, sd_hbm, ts_hbm, out_hbm,
               sc_v, sd_v, ts_v, out_v):
    wid = lax.axis_index("s") * 2 + lax.axis_index("c")
    per = NSAMP // NW
    base = wid * per
    pltpu.sync_copy(sc_hbm, sc_v)
    pltpu.sync_copy(sd_hbm, sd_v)
    pltpu.sync_copy(ts_hbm.at[pl.ds(base, per), pl.ds(0, 2)], ts_v)
    iota = lax.iota(jnp.int32, 16)
    zero_c = jnp.zeros((16,), jnp.int32)
    one_c = jnp.full((16,), 1, jnp.int32)

    def body(j, _):
        rows = j * 16 + iota
        i0 = plsc.load_gather(ts_v, [rows, zero_c])
        i1 = plsc.load_gather(ts_v, [rows, one_c])
        v = plsc.load_gather(sc_v, [i0]) + plsc.load_gather(sd_v, [i1])
        out_v[pl.ds(j * 16, 16)] = 1.0 / (1.0 + jnp.exp(-v))
        return 0
    lax.fori_loop(0, per // 16, body, 0)
    pltpu.sync_copy(out_v, out_hbm.at[pl.ds(base, per)])


_pair_kernel = functools.partial(
    pl.kernel,
    out_type=jax.ShapeDtypeStruct((NSAMP,), jnp.float32),
    mesh=plsc.VectorSubcoreMesh(**_MESH),
    compiler_params=pltpu.CompilerParams(needs_layout_passes=False),
    scratch_types=(
        pltpu.VMEM((CIRC,), jnp.float32),
        pltpu.VMEM((DIS,), jnp.float32),
        pltpu.VMEM((NSAMP // NW, 2), jnp.int32),
        pltpu.VMEM((NSAMP // NW,), jnp.float32),
    ),
)(_pair_body)


def kernel(circRNA, disease, cc_edge_index, dd_edge_index, cd_edge_index,
           tran_sample, lin_c_W, lin_d_W, cc_W0, cc_b0, cc_W1, cc_b1,
           dd_W0, dd_b0, dd_W1, dd_b1, cd_W0, cd_b0, cd_W1, cd_b1,
           mlp_W, mlp_b):
    acc, add_, acd = _adj_kernel(cc_edge_index, dd_edge_index,
                                 cd_edge_index)
    acc = acc.reshape(CIRC, CIRC // 2)
    add_ = add_.reshape(DIS, DIS // 2)
    acd = acd.reshape(CD, CD // 2)

    score_c, score_d = pl.pallas_call(
        _dense_body,
        out_shape=(jax.ShapeDtypeStruct((CIRC, 1), jnp.float32),
                   jax.ShapeDtypeStruct((DIS, 1), jnp.float32)),
    )(acc, add_, acd, circRNA, disease, lin_c_W, lin_d_W,
      cc_W0, cc_b0.reshape(1, -1), cc_W1, cc_b1.reshape(1, -1),
      dd_W0, dd_b0.reshape(1, -1), dd_W1, dd_b1.reshape(1, -1),
      cd_W0, cd_b0.reshape(1, -1), cd_W1, cd_b1.reshape(1, -1),
      mlp_W, mlp_b.reshape(1, 1))

    out = _pair_kernel(score_c.reshape(CIRC), score_d.reshape(DIS),
                       tran_sample)
    return out.reshape(NSAMP, 1)


# speedup vs baseline: 44.5269x; 1.0096x over previous
import functools

import jax
import jax.numpy as jnp
from jax import lax
from jax.experimental import pallas as pl
from jax.experimental.pallas import tpu as pltpu
from jax.experimental.pallas import tpu_sc as plsc

CIRC = 1024
DIS = 512
CD = CIRC + DIS
NW = 32
ECC = 65536
EDD = 32768
ECD = 98304
NSAMP = 16384

_MESH = dict(core_axis_name="c", subcore_axis_name="s", num_cores=2,
             num_subcores=16)


_TCC = ECC // 16
_TDD = EDD // 16
_TCD = ECD // 16
_NGRP = (_TCC + _TDD + _TCD) // 16
_NROW = _NGRP // 8
_RCC = _TCC // 128
_RDD = _TDD // 128
_RCD = _TCD // 128
_TAIL = 2048
_ZCH = 8192


def _adj_body(cc_ei, dd_ei, cd_ei,
              acc_out, add_out, acd_out,
              es_cc, ed_cc, es_dd, ed_dd, es_cd, ed_cd,
              idxb, valb, zbuf,
              scc, sdd, scd,
              lsem, zsem, ssem, osem):
    c = lax.axis_index("c")
    sid = lax.axis_index("s")
    zz = jnp.zeros((16,), jnp.int32)

    for hbm, row, vmem, cnt in ((cc_ei, 0, es_cc, _TCC), (cc_ei, 1, ed_cc, _TCC),
                                (dd_ei, 0, es_dd, _TDD), (dd_ei, 1, ed_dd, _TDD),
                                (cd_ei, 0, es_cd, _TCD), (cd_ei, 1, ed_cd, _TCD)):
        pltpu.async_copy(hbm.at[pl.ds(row, 1), pl.ds(sid * cnt, cnt)],
                         vmem, lsem)

    def zb(i, _):
        for u in range(8):
            zbuf[pl.ds((i * 8 + u) * 16, 16)] = zz
        return 0
    lax.fori_loop(0, _ZCH // 128, zb, 0)

    def zero_plan(half, n):
        share = (half * (n // 2) + _TAIL) // 16
        chunks = []
        done = 0
        while done < share:
            sz = min(_ZCH, share - done)
            chunks.append((done, sz))
            done += sz
        return share, chunks

    for shared, half, n in ((scc, CIRC // 2, CIRC), (sdd, DIS // 2, DIS),
                            (scd, CD // 2, CD)):
        share, chunks = zero_plan(half, n)
        for off, sz in chunks:
            pltpu.async_copy(zbuf.at[pl.ds(0, sz)],
                             shared.at[pl.ds(sid * share + off, sz)], zsem)

    for hbm, row, vmem, cnt in ((cc_ei, 0, es_cc, _TCC), (cc_ei, 1, ed_cc, _TCC),
                                (dd_ei, 0, es_dd, _TDD), (dd_ei, 1, ed_dd, _TDD),
                                (cd_ei, 0, es_cd, _TCD), (cd_ei, 1, ed_cd, _TCD)):
        pltpu.make_async_copy(hbm.at[pl.ds(row, 1), pl.ds(0, cnt)], vmem,
                              lsem).wait()

    def scan(es, ed, n, half, row0, nrows):
        base = c * half
        nw = n // 2
        dump = half * nw

        def body(r, _):
            for u in range(8):
                off = r * 128 + u * 16
                s16 = es[0, pl.ds(off, 16)]
                d16 = ed[0, pl.ds(off, 16)]
                m = (d16 >= base) & (d16 < base + half)
                hi_half = s16 >= nw
                sp = jnp.where(hi_half, s16 - nw, s16)
                val = jnp.where(hi_half, 65536, 1)
                idx = jnp.where(m, (d16 - base) * nw + sp, dump)
                idxb[pl.ds((row0 + r) * 128 + u * 16, 16)] = idx
                valb[pl.ds((row0 + r) * 128 + u * 16, 16)] = val
            return 0
        lax.fori_loop(0, nrows, body, 0)

    scan(es_cc, ed_cc, CIRC, CIRC // 2, 0, _RCC)
    scan(es_dd, ed_dd, DIS, DIS // 2, _RCC, _RDD)
    scan(es_cd, ed_cd, CD, CD // 2, _RCC + _RDD, _RCD)

    for shared, half, n in ((scc, CIRC // 2, CIRC), (sdd, DIS // 2, DIS),
                            (scd, CD // 2, CD)):
        share, chunks = zero_plan(half, n)
        for off, sz in chunks:
            pltpu.make_async_copy(zbuf.at[pl.ds(0, sz)],
                                  shared.at[pl.ds(0, sz)], zsem).wait()
    plsc.subcore_barrier()

    for shared, row0, nrows in ((scc, 0, _RCC), (sdd, _RCC, _RDD),
                                (scd, _RCC + _RDD, _RCD)):
        sl = pl.ds(row0 * 128, nrows * 128)
        pltpu.async_copy(valb.at[sl], shared.at[idxb.at[sl]], ssem,
                         add=True)
    for shared, row0, nrows in ((scc, 0, _RCC), (sdd, _RCC, _RDD),
                                (scd, _RCC + _RDD, _RCD)):
        sl = pl.ds(row0 * 128, nrows * 128)
        pltpu.make_async_copy(valb.at[sl], shared.at[idxb.at[sl]],
                              ssem).wait()
    plsc.subcore_barrier()

    for shared, half, n, out in ((scc, CIRC // 2, CIRC, acc_out),
                                 (sdd, DIS // 2, DIS, add_out),
                                 (scd, CD // 2, CD, acd_out)):
        share = half * (n // 2) // 16
        pltpu.async_copy(shared.at[pl.ds(sid * share, share)],
                         out.at[pl.ds(c * half * (n // 2) + sid * share,
                                      share)], osem)
    for shared, half, n, out in ((scc, CIRC // 2, CIRC, acc_out),
                                 (sdd, DIS // 2, DIS, add_out),
                                 (scd, CD // 2, CD, acd_out)):
        share = half * (n // 2) // 16
        pltpu.make_async_copy(shared.at[pl.ds(0, share)],
                              out.at[pl.ds(0, share)], osem).wait()


_adj_kernel = functools.partial(
    pl.kernel,
    out_type=(
        jax.ShapeDtypeStruct((CIRC * (CIRC // 2),), jnp.int32),
        jax.ShapeDtypeStruct((DIS * (DIS // 2),), jnp.int32),
        jax.ShapeDtypeStruct((CD * (CD // 2),), jnp.int32),
    ),
    mesh=plsc.VectorSubcoreMesh(**_MESH),
    compiler_params=pltpu.CompilerParams(needs_layout_passes=False),
    scratch_types=(
        pltpu.VMEM((1, _TCC), jnp.int32),
        pltpu.VMEM((1, _TCC), jnp.int32),
        pltpu.VMEM((1, _TDD), jnp.int32),
        pltpu.VMEM((1, _TDD), jnp.int32),
        pltpu.VMEM((1, _TCD), jnp.int32),
        pltpu.VMEM((1, _TCD), jnp.int32),
        pltpu.VMEM((_NROW * 128,), jnp.int32),
        pltpu.VMEM((_NROW * 128,), jnp.int32),
        pltpu.VMEM((_ZCH,), jnp.int32),
        pltpu.VMEM_SHARED(((CIRC // 2) * (CIRC // 2) + _TAIL,), jnp.int32),
        pltpu.VMEM_SHARED(((DIS // 2) * (DIS // 2) + _TAIL,), jnp.int32),
        pltpu.VMEM_SHARED(((CD // 2) * (CD // 2) + _TAIL,), jnp.int32),
        pltpu.SemaphoreType.DMA,
        pltpu.SemaphoreType.DMA,
        pltpu.SemaphoreType.DMA,
        pltpu.SemaphoreType.DMA,
    ),
)(_adj_body)


def _mm(a, b):
    return lax.dot_general(a, b, (((1,), (0,)), ((), ())),
                           preferred_element_type=jnp.float32)


def _gcn(P, feat, W0, b0, W1, b1):
    lo = (P & 0xFFFF).astype(jnp.float32)
    hi = lax.shift_right_logical(P, 16).astype(jnp.float32)
    A = jnp.concatenate([lo, hi], axis=1)
    deg = jnp.maximum(jnp.sum(A, axis=0), 1.0)
    norm = lax.rsqrt(deg)
    Ahat = A * norm[:, None] * norm[None, :]
    g0 = _mm(feat, W0)
    g1 = _mm(Ahat, g0)
    g2 = _mm(Ahat, g1)
    h0 = jnp.maximum(g0 + b0, 0.0)
    h1 = jnp.maximum(g1 + b0, 0.0)
    h2 = jnp.maximum(g2 + b0, 0.0)
    u0 = _mm(h0, W1[0:128]) + _mm(h1, W1[128:256]) + _mm(h2, W1[256:384])
    u1 = _mm(Ahat, u0)
    u2 = _mm(Ahat, u1)
    return (jnp.maximum(u0 + b1, 0.0), jnp.maximum(u1 + b1, 0.0),
            jnp.maximum(u2 + b1, 0.0))


def _dense_body(acc, add_, acd, circ, dis, lin_c, lin_d,
                cc_W0, cc_b0, cc_W1, cc_b1, dd_W0, dd_b0, dd_W1, dd_b1,
                cd_W0, cd_b0, cd_W1, cd_b1, mlp_W, mlp_b,
                sc_out, sd_out):
    cc = _gcn(acc[...], circ[...], cc_W0[...], cc_b0[...], cc_W1[...],
              cc_b1[...])
    dd = _gcn(add_[...], dis[...], dd_W0[...], dd_b0[...], dd_W1[...],
              dd_b1[...])
    cd_feat = jnp.concatenate(
        [_mm(circ[...], lin_c[...]), _mm(dis[...], lin_d[...])], axis=0)
    ass = _gcn(acd[...], cd_feat, cd_W0[...], cd_b0[...], cd_W1[...],
               cd_b1[...])
    W = mlp_W[...]
    score_c = _mm(cc[0], W[0:64]) + _mm(cc[1], W[64:128]) + _mm(cc[2], W[128:192])
    score_c = score_c + (_mm(ass[0][:CIRC], W[192:256]) +
                         _mm(ass[1][:CIRC], W[256:320]) +
                         _mm(ass[2][:CIRC], W[320:384]))
    score_d = _mm(dd[0], W[384:448]) + _mm(dd[1], W[448:512]) + _mm(dd[2], W[512:576])
    score_d = score_d + (_mm(ass[0][CIRC:], W[576:640]) +
                         _mm(ass[1][CIRC:], W[640:704]) +
                         _mm(ass[2][CIRC:], W[704:768]))
    sc_out[...] = score_c + mlp_b[...]
    sd_out[...] = score_d


def _pair_body(sc_hbm, sd_hbm, ts_hbm, out_hbm,
               sc_v, sd_v, ts_v, out_v):
    wid = lax.axis_index("s") * 2 + lax.axis_index("c")
    per = NSAMP // NW
    base = wid * per
    pltpu.sync_copy(sc_hbm, sc_v)
    pltpu.sync_copy(sd_hbm, sd_v)
    pltpu.sync_copy(ts_hbm.at[pl.ds(base, per), pl.ds(0, 2)], ts_v)
    iota = lax.iota(jnp.int32, 16)
    zero_c = jnp.zeros((16,), jnp.int32)
    one_c = jnp.full((16,), 1, jnp.int32)

    def body(j, _):
        rows = j * 16 + iota
        i0 = plsc.load_gather(ts_v, [rows, zero_c])
        i1 = plsc.load_gather(ts_v, [rows, one_c])
        v = plsc.load_gather(sc_v, [i0]) + plsc.load_gather(sd_v, [i1])
        out_v[pl.ds(j * 16, 16)] = 1.0 / (1.0 + jnp.exp(-v))
        return 0
    lax.fori_loop(0, per // 16, body, 0)
    pltpu.sync_copy(out_v, out_hbm.at[pl.ds(base, per)])


_pair_kernel = functools.partial(
    pl.kernel,
    out_type=jax.ShapeDtypeStruct((NSAMP,), jnp.float32),
    mesh=plsc.VectorSubcoreMesh(**_MESH),
    compiler_params=pltpu.CompilerParams(needs_layout_passes=False),
    scratch_types=(
        pltpu.VMEM((CIRC,), jnp.float32),
        pltpu.VMEM((DIS,), jnp.float32),
        pltpu.VMEM((NSAMP // NW, 2), jnp.int32),
        pltpu.VMEM((NSAMP // NW,), jnp.float32),
    ),
)(_pair_body)


def kernel(circRNA, disease, cc_edge_index, dd_edge_index, cd_edge_index,
           tran_sample, lin_c_W, lin_d_W, cc_W0, cc_b0, cc_W1, cc_b1,
           dd_W0, dd_b0, dd_W1, dd_b1, cd_W0, cd_b0, cd_W1, cd_b1,
           mlp_W, mlp_b):
    acc, add_, acd = _adj_kernel(cc_edge_index, dd_edge_index,
                                 cd_edge_index)
    acc = acc.reshape(CIRC, CIRC // 2)
    add_ = add_.reshape(DIS, DIS // 2)
    acd = acd.reshape(CD, CD // 2)

    score_c, score_d = pl.pallas_call(
        _dense_body,
        out_shape=(jax.ShapeDtypeStruct((CIRC, 1), jnp.float32),
                   jax.ShapeDtypeStruct((DIS, 1), jnp.float32)),
    )(acc, add_, acd, circRNA, disease, lin_c_W, lin_d_W,
      cc_W0, cc_b0.reshape(1, -1), cc_W1, cc_b1.reshape(1, -1),
      dd_W0, dd_b0.reshape(1, -1), dd_W1, dd_b1.reshape(1, -1),
      cd_W0, cd_b0.reshape(1, -1), cd_W1, cd_b1.reshape(1, -1),
      mlp_W, mlp_b.reshape(1, 1))

    out = _pair_kernel(score_c.reshape(CIRC), score_d.reshape(DIS),
                       tran_sample)
    return out.reshape(NSAMP, 1)


# speedup vs baseline: 73.5576x; 1.6520x over previous
import functools

import jax
import jax.numpy as jnp
from jax import lax
from jax.experimental import pallas as pl
from jax.experimental.pallas import tpu as pltpu
from jax.experimental.pallas import tpu_sc as plsc

CIRC = 1024
DIS = 512
CD = CIRC + DIS
NW = 32
ECC = 65536
EDD = 32768
ECD = 98304
NSAMP = 16384

_MESH = dict(core_axis_name="c", subcore_axis_name="s", num_cores=2,
             num_subcores=16)


_TCC = ECC // 16
_TDD = EDD // 16
_TCD = ECD // 16
_NGRP = (_TCC + _TDD + _TCD) // 16
_NROW = _NGRP // 8
_RCC = _TCC // 128
_RDD = _TDD // 128
_RCD = _TCD // 128
_TAIL = 2048
_ZCH = 8192


def _adj_body(cc_ei, dd_ei, cd_ei,
              acc_out, add_out, acd_out,
              es_cc, ed_cc, es_dd, ed_dd, es_cd, ed_cd,
              idxb, valb, zbuf,
              scc, sdd, scd,
              lsem, zsem, ssem, osem):
    c = lax.axis_index("c")
    sid = lax.axis_index("s")
    zz = jnp.zeros((16,), jnp.int32)

    for hbm, row, vmem, cnt in ((cc_ei, 0, es_cc, _TCC), (cc_ei, 1, ed_cc, _TCC),
                                (dd_ei, 0, es_dd, _TDD), (dd_ei, 1, ed_dd, _TDD),
                                (cd_ei, 0, es_cd, _TCD), (cd_ei, 1, ed_cd, _TCD)):
        pltpu.async_copy(hbm.at[pl.ds(row, 1), pl.ds(sid * cnt, cnt)],
                         vmem, lsem)

    def zb(i, _):
        for u in range(8):
            zbuf[pl.ds((i * 8 + u) * 16, 16)] = zz
        return 0
    lax.fori_loop(0, _ZCH // 128, zb, 0)

    def zero_plan(half, n):
        share = (half * (n // 2) + _TAIL) // 16
        chunks = []
        done = 0
        while done < share:
            sz = min(_ZCH, share - done)
            chunks.append((done, sz))
            done += sz
        return share, chunks

    for shared, half, n in ((scc, CIRC // 2, CIRC), (sdd, DIS // 2, DIS),
                            (scd, CD // 2, CD)):
        share, chunks = zero_plan(half, n)
        for off, sz in chunks:
            pltpu.async_copy(zbuf.at[pl.ds(0, sz)],
                             shared.at[pl.ds(sid * share + off, sz)], zsem)

    for hbm, row, vmem, cnt in ((cc_ei, 0, es_cc, _TCC), (cc_ei, 1, ed_cc, _TCC),
                                (dd_ei, 0, es_dd, _TDD), (dd_ei, 1, ed_dd, _TDD),
                                (cd_ei, 0, es_cd, _TCD), (cd_ei, 1, ed_cd, _TCD)):
        pltpu.make_async_copy(hbm.at[pl.ds(row, 1), pl.ds(0, cnt)], vmem,
                              lsem).wait()

    def scan(es, ed, n, half, row0, nrows):
        base = c * half
        nw = n // 2
        dump = half * nw

        def body(r, _):
            for u in range(8):
                off = r * 128 + u * 16
                s16 = es[0, pl.ds(off, 16)]
                d16 = ed[0, pl.ds(off, 16)]
                m = (d16 >= base) & (d16 < base + half)
                hi_half = s16 >= nw
                sp = jnp.where(hi_half, s16 - nw, s16)
                val = jnp.where(hi_half, 65536, 1)
                idx = jnp.where(m, (d16 - base) * nw + sp,
                                dump + (s16 & 1023))
                idxb[pl.ds((row0 + r) * 128 + u * 16, 16)] = idx
                valb[pl.ds((row0 + r) * 128 + u * 16, 16)] = val
            return 0
        lax.fori_loop(0, nrows, body, 0)

    scan(es_cc, ed_cc, CIRC, CIRC // 2, 0, _RCC)
    scan(es_dd, ed_dd, DIS, DIS // 2, _RCC, _RDD)
    scan(es_cd, ed_cd, CD, CD // 2, _RCC + _RDD, _RCD)

    for shared, half, n in ((scc, CIRC // 2, CIRC), (sdd, DIS // 2, DIS),
                            (scd, CD // 2, CD)):
        share, chunks = zero_plan(half, n)
        for off, sz in chunks:
            pltpu.make_async_copy(zbuf.at[pl.ds(0, sz)],
                                  shared.at[pl.ds(0, sz)], zsem).wait()
    plsc.subcore_barrier()

    for shared, row0, nrows in ((scc, 0, _RCC), (sdd, _RCC, _RDD),
                                (scd, _RCC + _RDD, _RCD)):
        sl = pl.ds(row0 * 128, nrows * 128)
        pltpu.async_copy(valb.at[sl], shared.at[idxb.at[sl]], ssem,
                         add=True)
    for shared, row0, nrows in ((scc, 0, _RCC), (sdd, _RCC, _RDD),
                                (scd, _RCC + _RDD, _RCD)):
        sl = pl.ds(row0 * 128, nrows * 128)
        pltpu.make_async_copy(valb.at[sl], shared.at[idxb.at[sl]],
                              ssem).wait()
    plsc.subcore_barrier()

    for shared, half, n, out in ((scc, CIRC // 2, CIRC, acc_out),
                                 (sdd, DIS // 2, DIS, add_out),
                                 (scd, CD // 2, CD, acd_out)):
        share = half * (n // 2) // 16
        pltpu.async_copy(shared.at[pl.ds(sid * share, share)],
                         out.at[pl.ds(c * half * (n // 2) + sid * share,
                                      share)], osem)
    for shared, half, n, out in ((scc, CIRC // 2, CIRC, acc_out),
                                 (sdd, DIS // 2, DIS, add_out),
                                 (scd, CD // 2, CD, acd_out)):
        share = half * (n // 2) // 16
        pltpu.make_async_copy(shared.at[pl.ds(0, share)],
                              out.at[pl.ds(0, share)], osem).wait()


_adj_kernel = functools.partial(
    pl.kernel,
    out_type=(
        jax.ShapeDtypeStruct((CIRC * (CIRC // 2),), jnp.int32),
        jax.ShapeDtypeStruct((DIS * (DIS // 2),), jnp.int32),
        jax.ShapeDtypeStruct((CD * (CD // 2),), jnp.int32),
    ),
    mesh=plsc.VectorSubcoreMesh(**_MESH),
    compiler_params=pltpu.CompilerParams(needs_layout_passes=False),
    scratch_types=(
        pltpu.VMEM((1, _TCC), jnp.int32),
        pltpu.VMEM((1, _TCC), jnp.int32),
        pltpu.VMEM((1, _TDD), jnp.int32),
        pltpu.VMEM((1, _TDD), jnp.int32),
        pltpu.VMEM((1, _TCD), jnp.int32),
        pltpu.VMEM((1, _TCD), jnp.int32),
        pltpu.VMEM((_NROW * 128,), jnp.int32),
        pltpu.VMEM((_NROW * 128,), jnp.int32),
        pltpu.VMEM((_ZCH,), jnp.int32),
        pltpu.VMEM_SHARED(((CIRC // 2) * (CIRC // 2) + _TAIL,), jnp.int32),
        pltpu.VMEM_SHARED(((DIS // 2) * (DIS // 2) + _TAIL,), jnp.int32),
        pltpu.VMEM_SHARED(((CD // 2) * (CD // 2) + _TAIL,), jnp.int32),
        pltpu.SemaphoreType.DMA,
        pltpu.SemaphoreType.DMA,
        pltpu.SemaphoreType.DMA,
        pltpu.SemaphoreType.DMA,
    ),
)(_adj_body)


def _mm(a, b):
    return lax.dot_general(a, b, (((1,), (0,)), ((), ())),
                           preferred_element_type=jnp.float32)


def _gcn(P, feat, W0, b0, W1, b1):
    lo = (P & 0xFFFF).astype(jnp.float32)
    hi = lax.shift_right_logical(P, 16).astype(jnp.float32)
    A = jnp.concatenate([lo, hi], axis=1)
    deg = jnp.maximum(jnp.sum(A, axis=0), 1.0)
    norm = lax.rsqrt(deg)
    Ahat = A * norm[:, None] * norm[None, :]
    g0 = _mm(feat, W0)
    g1 = _mm(Ahat, g0)
    g2 = _mm(Ahat, g1)
    h0 = jnp.maximum(g0 + b0, 0.0)
    h1 = jnp.maximum(g1 + b0, 0.0)
    h2 = jnp.maximum(g2 + b0, 0.0)
    u0 = _mm(h0, W1[0:128]) + _mm(h1, W1[128:256]) + _mm(h2, W1[256:384])
    u1 = _mm(Ahat, u0)
    u2 = _mm(Ahat, u1)
    return (jnp.maximum(u0 + b1, 0.0), jnp.maximum(u1 + b1, 0.0),
            jnp.maximum(u2 + b1, 0.0))


def _dense_body(acc, add_, acd, circ, dis, lin_c, lin_d,
                cc_W0, cc_b0, cc_W1, cc_b1, dd_W0, dd_b0, dd_W1, dd_b1,
                cd_W0, cd_b0, cd_W1, cd_b1, mlp_W, mlp_b,
                sc_out, sd_out):
    cc = _gcn(acc[...], circ[...], cc_W0[...], cc_b0[...], cc_W1[...],
              cc_b1[...])
    dd = _gcn(add_[...], dis[...], dd_W0[...], dd_b0[...], dd_W1[...],
              dd_b1[...])
    cd_feat = jnp.concatenate(
        [_mm(circ[...], lin_c[...]), _mm(dis[...], lin_d[...])], axis=0)
    ass = _gcn(acd[...], cd_feat, cd_W0[...], cd_b0[...], cd_W1[...],
               cd_b1[...])
    W = mlp_W[...]
    score_c = _mm(cc[0], W[0:64]) + _mm(cc[1], W[64:128]) + _mm(cc[2], W[128:192])
    score_c = score_c + (_mm(ass[0][:CIRC], W[192:256]) +
                         _mm(ass[1][:CIRC], W[256:320]) +
                         _mm(ass[2][:CIRC], W[320:384]))
    score_d = _mm(dd[0], W[384:448]) + _mm(dd[1], W[448:512]) + _mm(dd[2], W[512:576])
    score_d = score_d + (_mm(ass[0][CIRC:], W[576:640]) +
                         _mm(ass[1][CIRC:], W[640:704]) +
                         _mm(ass[2][CIRC:], W[704:768]))
    sc_out[...] = score_c + mlp_b[...]
    sd_out[...] = score_d


def _pair_body(sc_hbm, sd_hbm, ts_hbm, out_hbm,
               sc_v, sd_v, ts_v, out_v):
    wid = lax.axis_index("s") * 2 + lax.axis_index("c")
    per = NSAMP // NW
    base = wid * per
    pltpu.sync_copy(sc_hbm, sc_v)
    pltpu.sync_copy(sd_hbm, sd_v)
    pltpu.sync_copy(ts_hbm.at[pl.ds(base, per), pl.ds(0, 2)], ts_v)
    iota = lax.iota(jnp.int32, 16)
    zero_c = jnp.zeros((16,), jnp.int32)
    one_c = jnp.full((16,), 1, jnp.int32)

    def body(j, _):
        rows = j * 16 + iota
        i0 = plsc.load_gather(ts_v, [rows, zero_c])
        i1 = plsc.load_gather(ts_v, [rows, one_c])
        v = plsc.load_gather(sc_v, [i0]) + plsc.load_gather(sd_v, [i1])
        out_v[pl.ds(j * 16, 16)] = 1.0 / (1.0 + jnp.exp(-v))
        return 0
    lax.fori_loop(0, per // 16, body, 0)
    pltpu.sync_copy(out_v, out_hbm.at[pl.ds(base, per)])


_pair_kernel = functools.partial(
    pl.kernel,
    out_type=jax.ShapeDtypeStruct((NSAMP,), jnp.float32),
    mesh=plsc.VectorSubcoreMesh(**_MESH),
    compiler_params=pltpu.CompilerParams(needs_layout_passes=False),
    scratch_types=(
        pltpu.VMEM((CIRC,), jnp.float32),
        pltpu.VMEM((DIS,), jnp.float32),
        pltpu.VMEM((NSAMP // NW, 2), jnp.int32),
        pltpu.VMEM((NSAMP // NW,), jnp.float32),
    ),
)(_pair_body)


def kernel(circRNA, disease, cc_edge_index, dd_edge_index, cd_edge_index,
           tran_sample, lin_c_W, lin_d_W, cc_W0, cc_b0, cc_W1, cc_b1,
           dd_W0, dd_b0, dd_W1, dd_b1, cd_W0, cd_b0, cd_W1, cd_b1,
           mlp_W, mlp_b):
    acc, add_, acd = _adj_kernel(cc_edge_index, dd_edge_index,
                                 cd_edge_index)
    acc = acc.reshape(CIRC, CIRC // 2)
    add_ = add_.reshape(DIS, DIS // 2)
    acd = acd.reshape(CD, CD // 2)

    score_c, score_d = pl.pallas_call(
        _dense_body,
        out_shape=(jax.ShapeDtypeStruct((CIRC, 1), jnp.float32),
                   jax.ShapeDtypeStruct((DIS, 1), jnp.float32)),
    )(acc, add_, acd, circRNA, disease, lin_c_W, lin_d_W,
      cc_W0, cc_b0.reshape(1, -1), cc_W1, cc_b1.reshape(1, -1),
      dd_W0, dd_b0.reshape(1, -1), dd_W1, dd_b1.reshape(1, -1),
      cd_W0, cd_b0.reshape(1, -1), cd_W1, cd_b1.reshape(1, -1),
      mlp_W, mlp_b.reshape(1, 1))

    out = _pair_kernel(score_c.reshape(CIRC), score_d.reshape(DIS),
                       tran_sample)
    return out.reshape(NSAMP, 1)


# speedup vs baseline: 77.6152x; 1.0552x over previous
import functools

import jax
import jax.numpy as jnp
from jax import lax
from jax.experimental import pallas as pl
from jax.experimental.pallas import tpu as pltpu
from jax.experimental.pallas import tpu_sc as plsc

CIRC = 1024
DIS = 512
CD = CIRC + DIS
NW = 32
ECC = 65536
EDD = 32768
ECD = 98304
NSAMP = 16384

_MESH = dict(core_axis_name="c", subcore_axis_name="s", num_cores=2,
             num_subcores=16)


_TCC = ECC // 16
_TDD = EDD // 16
_TCD = ECD // 16
_NGRP = (_TCC + _TDD + _TCD) // 16
_NROW = _NGRP // 8
_RCC = _TCC // 128
_RDD = _TDD // 128
_RCD = _TCD // 128
_TAIL = 2048
_ZCH = 8192


def _adj_body(cc_ei, dd_ei, cd_ei,
              acc_out, add_out, acd_out,
              es_cc, ed_cc, es_dd, ed_dd, es_cd, ed_cd,
              idxb, valb, zbuf,
              scc, sdd, scd,
              lsem, zsem, ssem, osem):
    c = lax.axis_index("c")
    sid = lax.axis_index("s")
    zz = jnp.zeros((16,), jnp.int32)

    for hbm, row, vmem, cnt in ((cc_ei, 0, es_cc, _TCC), (cc_ei, 1, ed_cc, _TCC),
                                (dd_ei, 0, es_dd, _TDD), (dd_ei, 1, ed_dd, _TDD),
                                (cd_ei, 0, es_cd, _TCD), (cd_ei, 1, ed_cd, _TCD)):
        pltpu.async_copy(hbm.at[pl.ds(row, 1), pl.ds(sid * cnt, cnt)],
                         vmem, lsem)

    def zb(i, _):
        for u in range(8):
            zbuf[pl.ds((i * 8 + u) * 16, 16)] = zz
        return 0
    lax.fori_loop(0, _ZCH // 128, zb, 0)

    def zero_plan(half, n):
        share = (half * (n // 2) + _TAIL) // 16
        chunks = []
        done = 0
        while done < share:
            sz = min(_ZCH, share - done)
            chunks.append((done, sz))
            done += sz
        return share, chunks

    for shared, half, n in ((scc, CIRC // 2, CIRC), (sdd, DIS // 2, DIS),
                            (scd, CD // 2, CD)):
        share, chunks = zero_plan(half, n)
        for off, sz in chunks:
            pltpu.async_copy(zbuf.at[pl.ds(0, sz)],
                             shared.at[pl.ds(sid * share + off, sz)], zsem)

    for hbm, row, vmem, cnt in ((cc_ei, 0, es_cc, _TCC), (cc_ei, 1, ed_cc, _TCC),
                                (dd_ei, 0, es_dd, _TDD), (dd_ei, 1, ed_dd, _TDD),
                                (cd_ei, 0, es_cd, _TCD), (cd_ei, 1, ed_cd, _TCD)):
        pltpu.make_async_copy(hbm.at[pl.ds(row, 1), pl.ds(0, cnt)], vmem,
                              lsem).wait()

    def scan(es, ed, n, half, row0, nrows):
        base = c * half
        nw = n // 2
        dump = half * nw

        def body(r, _):
            for u in range(8):
                off = r * 128 + u * 16
                s16 = es[0, pl.ds(off, 16)]
                d16 = ed[0, pl.ds(off, 16)]
                m = (d16 >= base) & (d16 < base + half)
                hi_half = s16 >= nw
                sp = jnp.where(hi_half, s16 - nw, s16)
                val = jnp.where(hi_half, 65536, 1)
                idx = jnp.where(m, (d16 - base) * nw + sp,
                                dump + (s16 & 1023))
                idxb[pl.ds((row0 + r) * 128 + u * 16, 16)] = idx
                valb[pl.ds((row0 + r) * 128 + u * 16, 16)] = val
            return 0
        lax.fori_loop(0, nrows, body, 0)

    scan(es_cc, ed_cc, CIRC, CIRC // 2, 0, _RCC)
    scan(es_dd, ed_dd, DIS, DIS // 2, _RCC, _RDD)
    scan(es_cd, ed_cd, CD, CD // 2, _RCC + _RDD, _RCD)

    for shared, half, n in ((scc, CIRC // 2, CIRC), (sdd, DIS // 2, DIS),
                            (scd, CD // 2, CD)):
        share, chunks = zero_plan(half, n)
        for off, sz in chunks:
            pltpu.make_async_copy(zbuf.at[pl.ds(0, sz)],
                                  shared.at[pl.ds(0, sz)], zsem).wait()
    plsc.subcore_barrier()

    for shared, row0, nrows in ((scc, 0, _RCC), (sdd, _RCC, _RDD),
                                (scd, _RCC + _RDD, _RCD)):
        sl = pl.ds(row0 * 128, nrows * 128)
        pltpu.async_copy(valb.at[sl], shared.at[idxb.at[sl]], ssem,
                         add=True)
    for shared, row0, nrows in ((scc, 0, _RCC), (sdd, _RCC, _RDD),
                                (scd, _RCC + _RDD, _RCD)):
        sl = pl.ds(row0 * 128, nrows * 128)
        pltpu.make_async_copy(valb.at[sl], shared.at[idxb.at[sl]],
                              ssem).wait()
    plsc.subcore_barrier()

    for shared, half, n, out in ((scc, CIRC // 2, CIRC, acc_out),
                                 (sdd, DIS // 2, DIS, add_out),
                                 (scd, CD // 2, CD, acd_out)):
        share = half * (n // 2) // 16
        pltpu.async_copy(shared.at[pl.ds(sid * share, share)],
                         out.at[pl.ds(c * half * (n // 2) + sid * share,
                                      share)], osem)
    for shared, half, n, out in ((scc, CIRC // 2, CIRC, acc_out),
                                 (sdd, DIS // 2, DIS, add_out),
                                 (scd, CD // 2, CD, acd_out)):
        share = half * (n // 2) // 16
        pltpu.make_async_copy(shared.at[pl.ds(0, share)],
                              out.at[pl.ds(0, share)], osem).wait()


_adj_kernel = functools.partial(
    pl.kernel,
    out_type=(
        jax.ShapeDtypeStruct((CIRC * (CIRC // 2),), jnp.int32),
        jax.ShapeDtypeStruct((DIS * (DIS // 2),), jnp.int32),
        jax.ShapeDtypeStruct((CD * (CD // 2),), jnp.int32),
    ),
    mesh=plsc.VectorSubcoreMesh(**_MESH),
    compiler_params=pltpu.CompilerParams(needs_layout_passes=False),
    scratch_types=(
        pltpu.VMEM((1, _TCC), jnp.int32),
        pltpu.VMEM((1, _TCC), jnp.int32),
        pltpu.VMEM((1, _TDD), jnp.int32),
        pltpu.VMEM((1, _TDD), jnp.int32),
        pltpu.VMEM((1, _TCD), jnp.int32),
        pltpu.VMEM((1, _TCD), jnp.int32),
        pltpu.VMEM((_NROW * 128,), jnp.int32),
        pltpu.VMEM((_NROW * 128,), jnp.int32),
        pltpu.VMEM((_ZCH,), jnp.int32),
        pltpu.VMEM_SHARED(((CIRC // 2) * (CIRC // 2) + _TAIL,), jnp.int32),
        pltpu.VMEM_SHARED(((DIS // 2) * (DIS // 2) + _TAIL,), jnp.int32),
        pltpu.VMEM_SHARED(((CD // 2) * (CD // 2) + _TAIL,), jnp.int32),
        pltpu.SemaphoreType.DMA,
        pltpu.SemaphoreType.DMA,
        pltpu.SemaphoreType.DMA,
        pltpu.SemaphoreType.DMA,
    ),
)(_adj_body)


def _mm(a, b):
    return lax.dot_general(a, b, (((1,), (0,)), ((), ())),
                           preferred_element_type=jnp.float32)


def _gcn(P, g0, b0, W1, b1):
    lo = (P & 0xFFFF).astype(jnp.float32)
    hi = lax.shift_right_logical(P, 16).astype(jnp.float32)
    A = jnp.concatenate([lo, hi], axis=1)
    deg = jnp.maximum(jnp.sum(A, axis=0), 1.0)
    norm = lax.rsqrt(deg)
    Ahat = A * norm[:, None] * norm[None, :]
    g1 = _mm(Ahat, g0)
    g2 = _mm(Ahat, g1)
    h0 = jnp.maximum(g0 + b0, 0.0)
    h1 = jnp.maximum(g1 + b0, 0.0)
    h2 = jnp.maximum(g2 + b0, 0.0)
    u0 = _mm(h0, W1[0:128]) + _mm(h1, W1[128:256]) + _mm(h2, W1[256:384])
    u1 = _mm(Ahat, u0)
    u2 = _mm(Ahat, u1)
    return (jnp.maximum(u0 + b1, 0.0), jnp.maximum(u1 + b1, 0.0),
            jnp.maximum(u2 + b1, 0.0))


def _proj_body(circ, dis, lin_c, lin_d, cc_W0, dd_W0, cd_W0,
               g0cc_out, g0dd_out, g0cd_out):
    circ_v = circ[...]
    dis_v = dis[...]
    cdw = cd_W0[...]
    g0cc_out[...] = _mm(circ_v, cc_W0[...])
    g0dd_out[...] = _mm(dis_v, dd_W0[...])
    pc = _mm(circ_v, lin_c[...])
    pd = _mm(dis_v, lin_d[...])
    g0cd_out[...] = jnp.concatenate([_mm(pc, cdw), _mm(pd, cdw)], axis=0)


def _dense_body(acc, add_, acd, g0cc, g0dd, g0cd,
                cc_b0, cc_W1, cc_b1, dd_b0, dd_W1, dd_b1,
                cd_b0, cd_W1, cd_b1, mlp_W, mlp_b,
                sc_out, sd_out):
    cc = _gcn(acc[...], g0cc[...], cc_b0[...], cc_W1[...], cc_b1[...])
    dd = _gcn(add_[...], g0dd[...], dd_b0[...], dd_W1[...], dd_b1[...])
    ass = _gcn(acd[...], g0cd[...], cd_b0[...], cd_W1[...], cd_b1[...])
    W = mlp_W[...]
    score_c = _mm(cc[0], W[0:64]) + _mm(cc[1], W[64:128]) + _mm(cc[2], W[128:192])
    score_c = score_c + (_mm(ass[0][:CIRC], W[192:256]) +
                         _mm(ass[1][:CIRC], W[256:320]) +
                         _mm(ass[2][:CIRC], W[320:384]))
    score_d = _mm(dd[0], W[384:448]) + _mm(dd[1], W[448:512]) + _mm(dd[2], W[512:576])
    score_d = score_d + (_mm(ass[0][CIRC:], W[576:640]) +
                         _mm(ass[1][CIRC:], W[640:704]) +
                         _mm(ass[2][CIRC:], W[704:768]))
    sc_out[...] = score_c + mlp_b[...]
    sd_out[...] = score_d


def _pair_body(sc_hbm, sd_hbm, ts_hbm, out_hbm,
               sc_v, sd_v, ts_v, out_v):
    wid = lax.axis_index("s") * 2 + lax.axis_index("c")
    per = NSAMP // NW
    base = wid * per
    pltpu.sync_copy(sc_hbm, sc_v)
    pltpu.sync_copy(sd_hbm, sd_v)
    pltpu.sync_copy(ts_hbm.at[pl.ds(base, per), pl.ds(0, 2)], ts_v)
    iota = lax.iota(jnp.int32, 16)
    zero_c = jnp.zeros((16,), jnp.int32)
    one_c = jnp.full((16,), 1, jnp.int32)

    def body(j, _):
        rows = j * 16 + iota
        i0 = plsc.load_gather(ts_v, [rows, zero_c])
        i1 = plsc.load_gather(ts_v, [rows, one_c])
        v = plsc.load_gather(sc_v, [i0]) + plsc.load_gather(sd_v, [i1])
        out_v[pl.ds(j * 16, 16)] = 1.0 / (1.0 + jnp.exp(-v))
        return 0
    lax.fori_loop(0, per // 16, body, 0)
    pltpu.sync_copy(out_v, out_hbm.at[pl.ds(base, per)])


_pair_kernel = functools.partial(
    pl.kernel,
    out_type=jax.ShapeDtypeStruct((NSAMP,), jnp.float32),
    mesh=plsc.VectorSubcoreMesh(**_MESH),
    compiler_params=pltpu.CompilerParams(needs_layout_passes=False),
    scratch_types=(
        pltpu.VMEM((CIRC,), jnp.float32),
        pltpu.VMEM((DIS,), jnp.float32),
        pltpu.VMEM((NSAMP // NW, 2), jnp.int32),
        pltpu.VMEM((NSAMP // NW,), jnp.float32),
    ),
)(_pair_body)


def kernel(circRNA, disease, cc_edge_index, dd_edge_index, cd_edge_index,
           tran_sample, lin_c_W, lin_d_W, cc_W0, cc_b0, cc_W1, cc_b1,
           dd_W0, dd_b0, dd_W1, dd_b1, cd_W0, cd_b0, cd_W1, cd_b1,
           mlp_W, mlp_b):
    g0cc, g0dd, g0cd = pl.pallas_call(
        _proj_body,
        out_shape=(jax.ShapeDtypeStruct((CIRC, 128), jnp.float32),
                   jax.ShapeDtypeStruct((DIS, 128), jnp.float32),
                   jax.ShapeDtypeStruct((CD, 128), jnp.float32)),
    )(circRNA, disease, lin_c_W, lin_d_W, cc_W0, dd_W0, cd_W0)

    acc, add_, acd = _adj_kernel(cc_edge_index, dd_edge_index,
                                 cd_edge_index)
    acc = acc.reshape(CIRC, CIRC // 2)
    add_ = add_.reshape(DIS, DIS // 2)
    acd = acd.reshape(CD, CD // 2)

    score_c, score_d = pl.pallas_call(
        _dense_body,
        out_shape=(jax.ShapeDtypeStruct((CIRC, 1), jnp.float32),
                   jax.ShapeDtypeStruct((DIS, 1), jnp.float32)),
    )(acc, add_, acd, g0cc, g0dd, g0cd,
      cc_b0.reshape(1, -1), cc_W1, cc_b1.reshape(1, -1),
      dd_b0.reshape(1, -1), dd_W1, dd_b1.reshape(1, -1),
      cd_b0.reshape(1, -1), cd_W1, cd_b1.reshape(1, -1),
      mlp_W, mlp_b.reshape(1, 1))

    out = _pair_kernel(score_c.reshape(CIRC), score_d.reshape(DIS),
                       tran_sample)
    return out.reshape(NSAMP, 1)
